# baseline (speedup 1.0000x reference)
import functools

import jax
import jax.numpy as jnp
from jax import lax
from jax.experimental import pallas as pl
from jax.experimental.pallas import tpu as pltpu
from jax.experimental.pallas import tpu_sc as plsc

_N = 10000
_E = 320000
_G = 32
_F = 64
_NC = 2
_NS = 16
_NW = _NC * _NS
_CH = 128
_NCHUNK = 80
_CPW = _CH * _NCHUNK
_EPAD = _CPW * _NW
_TOTCH = _EPAD // _CH
_NCH0 = 80
_NCH1 = 80
_W8 = 8
_NPAD = 10240
_RPC = _NPAD // _NS
_DW = 8

_mesh = plsc.VectorSubcoreMesh(core_axis_name="c", subcore_axis_name="s")
_sc_params = pltpu.CompilerParams(use_tc_tiling_on_sc=False)


@functools.partial(
    pl.kernel,
    out_type=jax.ShapeDtypeStruct((_NC, _NPAD, _DW), jnp.float32),
    mesh=_mesh,
    compiler_params=_sc_params,
    scratch_types=[
        pltpu.VMEM((_NCHUNK, _CH), jnp.int32),
        pltpu.VMEM((_CH, _DW), jnp.float32),
        pltpu.VMEM((_RPC, _DW), jnp.float32),
        pltpu.VMEM_SHARED((_NPAD, _DW), jnp.float32),
        pltpu.SemaphoreType.DMA,
    ],
)
def _sc_degree(dst_hbm, ones_hbm, zeros_hbm, out_hbm, didx, ones, zb, acc,
               sem):
    c = lax.axis_index("c")
    s = lax.axis_index("s")
    wid = c * _NS + s

    pltpu.sync_copy(dst_hbm.at[pl.ds(wid * _NCHUNK, _NCHUNK)], didx)
    pltpu.sync_copy(ones_hbm, ones)
    pltpu.sync_copy(zeros_hbm, acc.at[pl.ds(s * _RPC, _RPC)])
    plsc.subcore_barrier()

    @pl.loop(0, _NCHUNK, step=8)
    def _(j):
        for b in range(8):
            pltpu.async_copy(ones, acc.at[didx.at[j + b]], sem, add=True)
        for b in range(8):
            pltpu.make_async_copy(ones, acc.at[didx.at[j + b]], sem).wait()

    plsc.subcore_barrier()
    pltpu.sync_copy(acc.at[pl.ds(s * _RPC, _RPC)], zb)
    pltpu.sync_copy(zb, out_hbm.at[c].at[pl.ds(s * _RPC, _RPC)])


@functools.partial(
    pl.kernel,
    out_type=jax.ShapeDtypeStruct((_NC, _NPAD, _F), jnp.float32),
    mesh=_mesh,
    compiler_params=_sc_params,
    scratch_types=[
        pltpu.VMEM((2, _W8, _CH), jnp.int32),
        pltpu.VMEM((2, _W8, _CH), jnp.int32),
        pltpu.VMEM((8, _CH, _F), jnp.float32),
        pltpu.VMEM_SHARED((_NPAD, _F), jnp.float32),
        pltpu.SemaphoreType.DMA,
        pltpu.SemaphoreType.DMA,
        pltpu.SemaphoreType.DMA,
    ],
)
def _sc_propagate(table_hbm, src_hbm, dst_hbm, zeros_hbm, out_hbm,
                  sidx, didx, rows, acc, semg, semi, sems):
    c = lax.axis_index("c")
    s = lax.axis_index("s")

    wb = jnp.where(c == 0, s * _NCH0, _NS * _NCH0 + s * _NCH1)
    nwv = jnp.where(c == 0, _NCH0 // _W8, _NCH1 // _W8)

    pltpu.sync_copy(zeros_hbm, acc.at[pl.ds(s * _RPC, _RPC)])
    plsc.subcore_barrier()

    def _idx_load(wave, slot, sync=False):
        rb = wb + wave * _W8
        if sync:
            pltpu.sync_copy(src_hbm.at[pl.ds(rb, _W8)], sidx.at[slot])
            pltpu.sync_copy(dst_hbm.at[pl.ds(rb, _W8)], didx.at[slot])
        else:
            pltpu.async_copy(src_hbm.at[pl.ds(rb, _W8)], sidx.at[slot], semi)
            pltpu.async_copy(dst_hbm.at[pl.ds(rb, _W8)], didx.at[slot], semi)

    def _idx_drain(wave, slot):
        rb = wb + wave * _W8
        pltpu.make_async_copy(src_hbm.at[pl.ds(rb, _W8)], sidx.at[slot],
                              semi).wait()
        pltpu.make_async_copy(dst_hbm.at[pl.ds(rb, _W8)], didx.at[slot],
                              semi).wait()

    def _gather(slot, r, buf):
        pltpu.async_copy(table_hbm.at[sidx.at[slot, r]], rows.at[buf], semg)

    def _gdrain(slot, r, buf):
        pltpu.make_async_copy(table_hbm.at[sidx.at[slot, r]], rows.at[buf],
                              semg).wait()

    def _scatter(slot, r, buf):
        pltpu.async_copy(rows.at[buf], acc.at[didx.at[slot, r]], sems,
                         add=True)

    def _sdrain(slot, r, buf):
        pltpu.make_async_copy(rows.at[buf], acc.at[didx.at[slot, r]],
                              sems).wait()

    @pl.when(nwv > 0)
    def _():
        _idx_load(0, 0, sync=True)
        _idx_load(1, 1)
        for b in range(4):
            _gather(0, b, b)

    def _half(w, slot):
        other = 1 - slot

        @pl.when(w > 0)
        def _():
            for b in range(4):
                _sdrain(other, 4 + b, 4 + b)

        for b in range(4):
            _gather(slot, 4 + b, 4 + b)
        for b in range(4):
            _gdrain(slot, b, b)
        for b in range(4):
            _scatter(slot, b, b)

        @pl.when(w + 1 < nwv)
        def _():
            _idx_drain(w + 1, other)

        for b in range(4):
            _sdrain(slot, b, b)

        @pl.when(w + 1 < nwv)
        def _():
            for b in range(4):
                _gather(other, b, b)

        for b in range(4):
            _gdrain(slot, 4 + b, 4 + b)
        for b in range(4):
            _scatter(slot, 4 + b, 4 + b)

        @pl.when(w + 2 < nwv)
        def _():
            _idx_load(w + 2, slot)

    @pl.loop(0, nwv, step=2)
    def _(w):
        _half(w, 0)
        _half(w + 1, 1)

    @pl.when(nwv > 0)
    def _():
        for b in range(4):
            _sdrain(1, 4 + b, 4 + b)

    plsc.subcore_barrier()
    for k in range(_RPC // _CH):
        pltpu.sync_copy(acc.at[pl.ds(s * _RPC + k * _CH, _CH)], rows.at[0])
        pltpu.sync_copy(rows.at[0],
                        out_hbm.at[c].at[pl.ds(s * _RPC + k * _CH, _CH)])


def _tc1_body(dacc_ref, x_ref, w1_ref, g1_ref, dinv_ref):
    dacc = dacc_ref[...]
    indeg = dacc[0, :_N, 0:1] + dacc[1, :_N, 0:1]
    dinv = lax.rsqrt(indeg + 1.0)
    dinv_ref[...] = dinv
    h = jnp.dot(x_ref[...], w1_ref[...], preferred_element_type=jnp.float32)
    g1_ref[...] = dinv * h


def _tc2_body(acc_ref, g1_ref, dinv_ref, b1_ref, g2_ref):
    a = acc_ref[...]
    dinv = dinv_ref[...]
    pre = dinv * (a[0, :_N, :] + a[1, :_N, :] + g1_ref[...]) + b1_ref[...]
    h1 = jnp.maximum(pre, 0.0)
    g2_ref[...] = dinv * h1


def _tc3_body(acc_ref, g2_ref, dinv_ref, w2_ref, b2_ref, w3_ref, g3_ref):
    a = acc_ref[...]
    dinv = dinv_ref[...]
    p2 = dinv * (a[0, :_N, :] + a[1, :_N, :] + g2_ref[...])
    h2 = jnp.maximum(
        jnp.dot(p2, w2_ref[...], preferred_element_type=jnp.float32)
        + b2_ref[...], 0.0)
    g3_ref[...] = dinv * jnp.dot(h2, w3_ref[...],
                                 preferred_element_type=jnp.float32)


def _tc4_body(acc_ref, g3_ref, dinv_ref, b3_ref, batch_ref, wfc_ref, bfc_ref,
              out_ref):
    a = acc_ref[...]
    dinv = dinv_ref[...]
    pre = dinv * (a[0, :_N, :] + a[1, :_N, :] + g3_ref[...]) + b3_ref[...]
    h3 = jnp.maximum(pre, 0.0)
    seg = lax.broadcasted_iota(jnp.int32, (_G, _N), 0)
    onehot = (seg == batch_ref[...]).astype(jnp.float32)
    sums = jnp.dot(onehot, h3, preferred_element_type=jnp.float32)
    counts = jnp.sum(onehot, axis=1, keepdims=True)
    pooled = sums / jnp.maximum(counts, 1.0)
    logits = jnp.dot(pooled, wfc_ref[...],
                     preferred_element_type=jnp.float32) + bfc_ref[...]
    m = jnp.max(logits, axis=1, keepdims=True)
    shifted = logits - m
    lse = jnp.log(jnp.sum(jnp.exp(shifted), axis=1, keepdims=True))
    out_ref[...] = shifted - lse


def kernel(x, edge_index, batch, W1, b1, W2, b2, W3, b3, Wfc, bfc):
    src, dst = edge_index[0], edge_index[1]
    pad = _EPAD - _E
    pad_src = jnp.arange(pad, dtype=src.dtype) % _N
    pad_dst = _N + jnp.arange(pad, dtype=dst.dtype) % (_NPAD - _N)
    src_p = jnp.concatenate([src, pad_src])
    dst_p = jnp.concatenate([dst, pad_dst])
    src2 = src_p.reshape(_EPAD // _CH, _CH)
    dst2 = dst_p.reshape(_EPAD // _CH, _CH)
    batch2 = batch.reshape(1, _N)
    b1r, b2r, b3r = b1.reshape(1, -1), b2.reshape(1, -1), b3.reshape(1, -1)
    bfcr = bfc.reshape(1, -1)

    ones_d = jnp.ones((_CH, _DW), jnp.float32)
    zeros_d = jnp.zeros((_RPC, _DW), jnp.float32)
    zeros_f = jnp.zeros((_RPC, _F), jnp.float32)

    dacc = _sc_degree(dst2, ones_d, zeros_d)

    g1, dinv = pl.pallas_call(
        _tc1_body,
        out_shape=(jax.ShapeDtypeStruct((_N, 64), jnp.float32),
                   jax.ShapeDtypeStruct((_N, 1), jnp.float32)),
    )(dacc, x, W1)

    a1 = _sc_propagate(g1, src2, dst2, zeros_f)
    g2 = pl.pallas_call(
        _tc2_body,
        out_shape=jax.ShapeDtypeStruct((_N, 64), jnp.float32),
    )(a1, g1, dinv, b1r)

    a2 = _sc_propagate(g2, src2, dst2, zeros_f)
    g3 = pl.pallas_call(
        _tc3_body,
        out_shape=jax.ShapeDtypeStruct((_N, 64), jnp.float32),
    )(a2, g2, dinv, W2, b2r, W3)

    a3 = _sc_propagate(g3, src2, dst2, zeros_f)
    out = pl.pallas_call(
        _tc4_body,
        out_shape=jax.ShapeDtypeStruct((_G, 10), jnp.float32),
    )(a3, g3, dinv, b3r, batch2, Wfc, bfcr)
    return out


# speedup vs baseline: 1.0126x; 1.0126x over previous
import functools

import jax
import jax.numpy as jnp
from jax import lax
from jax.experimental import pallas as pl
from jax.experimental.pallas import tpu as pltpu
from jax.experimental.pallas import tpu_sc as plsc

_N = 10000
_E = 320000
_G = 32
_F = 64
_NC = 2
_NS = 16
_NW = _NC * _NS
_CH = 128
_NCHUNK = 80
_CPW = _CH * _NCHUNK
_EPAD = _CPW * _NW
_TOTCH = _EPAD // _CH
_NCH0 = 80
_NCH1 = 80
_W8 = 8
_NPAD = 10240
_RPC = _NPAD // _NS
_DW = 8

_mesh = plsc.VectorSubcoreMesh(core_axis_name="c", subcore_axis_name="s")
_sc_params = pltpu.CompilerParams(use_tc_tiling_on_sc=False)


@functools.partial(
    pl.kernel,
    out_type=jax.ShapeDtypeStruct((_NC, _NPAD, _DW), jnp.float32),
    mesh=_mesh,
    compiler_params=_sc_params,
    scratch_types=[
        pltpu.VMEM((_NCHUNK, _CH), jnp.int32),
        pltpu.VMEM((_CH, _DW), jnp.float32),
        pltpu.VMEM((_RPC, _DW), jnp.float32),
        pltpu.VMEM_SHARED((_NPAD, _DW), jnp.float32),
        pltpu.SemaphoreType.DMA,
    ],
)
def _sc_degree(dst_hbm, ones_hbm, zeros_hbm, out_hbm, didx, ones, zb, acc,
               sem):
    c = lax.axis_index("c")
    s = lax.axis_index("s")
    wid = c * _NS + s

    pltpu.sync_copy(dst_hbm.at[pl.ds(wid * _NCHUNK, _NCHUNK)], didx)
    pltpu.sync_copy(ones_hbm, ones)
    pltpu.sync_copy(zeros_hbm, acc.at[pl.ds(s * _RPC, _RPC)])
    plsc.subcore_barrier()

    @pl.loop(0, _NCHUNK, step=8)
    def _(j):
        for b in range(8):
            pltpu.async_copy(ones, acc.at[didx.at[j + b]], sem, add=True)
        for b in range(8):
            pltpu.make_async_copy(ones, acc.at[didx.at[j + b]], sem).wait()

    plsc.subcore_barrier()
    pltpu.sync_copy(acc.at[pl.ds(s * _RPC, _RPC)], zb)
    pltpu.sync_copy(zb, out_hbm.at[c].at[pl.ds(s * _RPC, _RPC)])


@functools.partial(
    pl.kernel,
    out_type=jax.ShapeDtypeStruct((_NC, _NPAD, _F), jnp.float32),
    mesh=_mesh,
    compiler_params=_sc_params,
    scratch_types=[
        pltpu.VMEM((2, _W8, _CH), jnp.int32),
        pltpu.VMEM((2, _W8, _CH), jnp.int32),
        pltpu.VMEM((8, _CH, _F), jnp.float32),
        pltpu.VMEM_SHARED((_NPAD, _F), jnp.float32),
        pltpu.SemaphoreType.DMA,
        pltpu.SemaphoreType.DMA,
    ],
)
def _sc_propagate(table_hbm, src_hbm, dst_hbm, zeros_hbm, out_hbm,
                  sidx, didx, rows, acc, semg, semi):
    c = lax.axis_index("c")
    s = lax.axis_index("s")

    wb = jnp.where(c == 0, s * _NCH0, _NS * _NCH0 + s * _NCH1)
    nwv = jnp.where(c == 0, _NCH0 // _W8, _NCH1 // _W8)

    pltpu.sync_copy(zeros_hbm, acc.at[pl.ds(s * _RPC, _RPC)])
    plsc.subcore_barrier()

    def _idx_load(wave, slot, sync=False):
        rb = wb + wave * _W8
        if sync:
            pltpu.sync_copy(src_hbm.at[pl.ds(rb, _W8)], sidx.at[slot])
            pltpu.sync_copy(dst_hbm.at[pl.ds(rb, _W8)], didx.at[slot])
        else:
            pltpu.async_copy(src_hbm.at[pl.ds(rb, _W8)], sidx.at[slot], semi)
            pltpu.async_copy(dst_hbm.at[pl.ds(rb, _W8)], didx.at[slot], semi)

    def _idx_drain(wave, slot):
        rb = wb + wave * _W8
        pltpu.make_async_copy(src_hbm.at[pl.ds(rb, _W8)], sidx.at[slot],
                              semi).wait()
        pltpu.make_async_copy(dst_hbm.at[pl.ds(rb, _W8)], didx.at[slot],
                              semi).wait()

    def _gather(slot, r, buf):
        pltpu.async_copy(table_hbm.at[sidx.at[slot, r]], rows.at[buf], semg)

    def _gdrain(slot, r, buf):
        pltpu.make_async_copy(table_hbm.at[sidx.at[slot, r]], rows.at[buf],
                              semg).wait()

    def _scatter(slot, r, buf):
        pltpu.sync_copy(rows.at[buf], acc.at[didx.at[slot, r]], add=True)

    @pl.when(nwv > 0)
    def _():
        _idx_load(0, 0, sync=True)
        _idx_load(1, 1)
        for b in range(4):
            _gather(0, b, b)

    def _half(w, slot):
        other = 1 - slot
        for b in range(4):
            _gather(slot, 4 + b, 4 + b)
        for b in range(4):
            _gdrain(slot, b, b)
        for b in range(4):
            _scatter(slot, b, b)

        @pl.when(w + 1 < nwv)
        def _():
            _idx_drain(w + 1, other)
            for b in range(4):
                _gather(other, b, b)

        for b in range(4):
            _gdrain(slot, 4 + b, 4 + b)
        for b in range(4):
            _scatter(slot, 4 + b, 4 + b)

        @pl.when(w + 2 < nwv)
        def _():
            _idx_load(w + 2, slot)

    @pl.loop(0, nwv, step=2)
    def _(w):
        _half(w, 0)
        _half(w + 1, 1)

    plsc.subcore_barrier()
    for k in range(_RPC // _CH):
        pltpu.sync_copy(acc.at[pl.ds(s * _RPC + k * _CH, _CH)], rows.at[0])
        pltpu.sync_copy(rows.at[0],
                        out_hbm.at[c].at[pl.ds(s * _RPC + k * _CH, _CH)])


def _tc1_body(dacc_ref, x_ref, w1_ref, g1_ref, dinv_ref):
    dacc = dacc_ref[...]
    indeg = dacc[0, :_N, 0:1] + dacc[1, :_N, 0:1]
    dinv = lax.rsqrt(indeg + 1.0)
    dinv_ref[...] = dinv
    h = jnp.dot(x_ref[...], w1_ref[...], preferred_element_type=jnp.float32)
    g1_ref[...] = dinv * h


def _tc2_body(acc_ref, g1_ref, dinv_ref, b1_ref, g2_ref):
    a = acc_ref[...]
    dinv = dinv_ref[...]
    pre = dinv * (a[0, :_N, :] + a[1, :_N, :] + g1_ref[...]) + b1_ref[...]
    h1 = jnp.maximum(pre, 0.0)
    g2_ref[...] = dinv * h1


def _tc3_body(acc_ref, g2_ref, dinv_ref, w2_ref, b2_ref, w3_ref, g3_ref):
    a = acc_ref[...]
    dinv = dinv_ref[...]
    p2 = dinv * (a[0, :_N, :] + a[1, :_N, :] + g2_ref[...])
    h2 = jnp.maximum(
        jnp.dot(p2, w2_ref[...], preferred_element_type=jnp.float32)
        + b2_ref[...], 0.0)
    g3_ref[...] = dinv * jnp.dot(h2, w3_ref[...],
                                 preferred_element_type=jnp.float32)


def _tc4_body(acc_ref, g3_ref, dinv_ref, b3_ref, batch_ref, wfc_ref, bfc_ref,
              out_ref):
    a = acc_ref[...]
    dinv = dinv_ref[...]
    pre = dinv * (a[0, :_N, :] + a[1, :_N, :] + g3_ref[...]) + b3_ref[...]
    h3 = jnp.maximum(pre, 0.0)
    seg = lax.broadcasted_iota(jnp.int32, (_G, _N), 0)
    onehot = (seg == batch_ref[...]).astype(jnp.float32)
    sums = jnp.dot(onehot, h3, preferred_element_type=jnp.float32)
    counts = jnp.sum(onehot, axis=1, keepdims=True)
    pooled = sums / jnp.maximum(counts, 1.0)
    logits = jnp.dot(pooled, wfc_ref[...],
                     preferred_element_type=jnp.float32) + bfc_ref[...]
    m = jnp.max(logits, axis=1, keepdims=True)
    shifted = logits - m
    lse = jnp.log(jnp.sum(jnp.exp(shifted), axis=1, keepdims=True))
    out_ref[...] = shifted - lse


def kernel(x, edge_index, batch, W1, b1, W2, b2, W3, b3, Wfc, bfc):
    src, dst = edge_index[0], edge_index[1]
    pad = _EPAD - _E
    pad_src = jnp.arange(pad, dtype=src.dtype) % _N
    pad_dst = _N + jnp.arange(pad, dtype=dst.dtype) % (_NPAD - _N)
    src_p = jnp.concatenate([src, pad_src])
    dst_p = jnp.concatenate([dst, pad_dst])
    src2 = src_p.reshape(_EPAD // _CH, _CH)
    dst2 = dst_p.reshape(_EPAD // _CH, _CH)
    batch2 = batch.reshape(1, _N)
    b1r, b2r, b3r = b1.reshape(1, -1), b2.reshape(1, -1), b3.reshape(1, -1)
    bfcr = bfc.reshape(1, -1)

    ones_d = jnp.ones((_CH, _DW), jnp.float32)
    zeros_d = jnp.zeros((_RPC, _DW), jnp.float32)
    zeros_f = jnp.zeros((_RPC, _F), jnp.float32)

    dacc = _sc_degree(dst2, ones_d, zeros_d)

    g1, dinv = pl.pallas_call(
        _tc1_body,
        out_shape=(jax.ShapeDtypeStruct((_N, 64), jnp.float32),
                   jax.ShapeDtypeStruct((_N, 1), jnp.float32)),
    )(dacc, x, W1)

    a1 = _sc_propagate(g1, src2, dst2, zeros_f)
    g2 = pl.pallas_call(
        _tc2_body,
        out_shape=jax.ShapeDtypeStruct((_N, 64), jnp.float32),
    )(a1, g1, dinv, b1r)

    a2 = _sc_propagate(g2, src2, dst2, zeros_f)
    g3 = pl.pallas_call(
        _tc3_body,
        out_shape=jax.ShapeDtypeStruct((_N, 64), jnp.float32),
    )(a2, g2, dinv, W2, b2r, W3)

    a3 = _sc_propagate(g3, src2, dst2, zeros_f)
    out = pl.pallas_call(
        _tc4_body,
        out_shape=jax.ShapeDtypeStruct((_G, 10), jnp.float32),
    )(a3, g3, dinv, b3r, batch2, Wfc, bfcr)
    return out


# speedup vs baseline: 1.0411x; 1.0281x over previous
import functools

import jax
import jax.numpy as jnp
from jax import lax
from jax.experimental import pallas as pl
from jax.experimental.pallas import tpu as pltpu
from jax.experimental.pallas import tpu_sc as plsc

_N = 10000
_E = 320000
_G = 32
_F = 64
_NC = 2
_NS = 16
_NW = _NC * _NS
_CH = 128
_NCHUNK = 80
_CPW = _CH * _NCHUNK
_EPAD = _CPW * _NW
_TOTCH = _EPAD // _CH
_NCH0 = 80
_NCH1 = 80
_W8 = 8
_NPAD = 10240
_RPC = _NPAD // _NS
_DW = 16

_mesh = plsc.VectorSubcoreMesh(core_axis_name="c", subcore_axis_name="s")
_sc_params = pltpu.CompilerParams(use_tc_tiling_on_sc=False)


@functools.partial(
    pl.kernel,
    out_type=jax.ShapeDtypeStruct((_NC, _NPAD, _DW), jnp.float32),
    mesh=_mesh,
    compiler_params=_sc_params,
    scratch_types=[
        pltpu.VMEM((_NCHUNK, _CH), jnp.int32),
        pltpu.VMEM((_CH, _DW), jnp.float32),
        pltpu.VMEM((_RPC, _DW), jnp.float32),
        pltpu.VMEM_SHARED((_NPAD, _DW), jnp.float32),
        pltpu.SemaphoreType.DMA,
    ],
)
def _sc_degree(dst_hbm, out_hbm, didx, ones, zb, acc, sem):
    c = lax.axis_index("c")
    s = lax.axis_index("s")
    wid = c * _NS + s
    ones_v = jnp.full((16,), 1.0, jnp.float32)
    zeros_v = jnp.zeros((16,), jnp.float32)

    pltpu.sync_copy(dst_hbm.at[pl.ds(wid * _NCHUNK, _NCHUNK)], didx)

    @pl.loop(0, _CH)
    def _(i):
        ones[i, :] = ones_v

    @pl.loop(0, _RPC)
    def _(i):
        zb[i, :] = zeros_v

    pltpu.sync_copy(zb, acc.at[pl.ds(s * _RPC, _RPC)])
    plsc.subcore_barrier()

    @pl.loop(0, _NCHUNK, step=8)
    def _(j):
        for b in range(8):
            pltpu.async_copy(ones, acc.at[didx.at[j + b]], sem, add=True)
        for b in range(8):
            pltpu.make_async_copy(ones, acc.at[didx.at[j + b]], sem).wait()

    plsc.subcore_barrier()
    pltpu.sync_copy(acc.at[pl.ds(s * _RPC, _RPC)], zb)
    pltpu.sync_copy(zb, out_hbm.at[c].at[pl.ds(s * _RPC, _RPC)])


@functools.partial(
    pl.kernel,
    out_type=jax.ShapeDtypeStruct((_NC, _NPAD, _F), jnp.float32),
    mesh=_mesh,
    compiler_params=_sc_params,
    scratch_types=[
        pltpu.VMEM((2, _W8, _CH), jnp.int32),
        pltpu.VMEM((2, _W8, _CH), jnp.int32),
        pltpu.VMEM((8, _CH, _F), jnp.float32),
        pltpu.VMEM_SHARED((_NPAD, _F), jnp.float32),
        pltpu.SemaphoreType.DMA,
        pltpu.SemaphoreType.DMA,
    ],
)
def _sc_propagate(table_hbm, src_hbm, dst_hbm, out_hbm,
                  sidx, didx, rows, acc, semg, semi):
    c = lax.axis_index("c")
    s = lax.axis_index("s")
    zeros_v = jnp.zeros((16,), jnp.float32)

    wb = jnp.where(c == 0, s * _NCH0, _NS * _NCH0 + s * _NCH1)
    nwv = jnp.where(c == 0, _NCH0 // _W8, _NCH1 // _W8)

    @pl.loop(0, _CH)
    def _(i):
        for k in range(_F // 16):
            rows[0, i, pl.ds(k * 16, 16)] = zeros_v

    for k in range(_RPC // _CH):
        pltpu.sync_copy(rows.at[0], acc.at[pl.ds(s * _RPC + k * _CH, _CH)])
    plsc.subcore_barrier()

    def _idx_load(wave, slot, sync=False):
        rb = wb + wave * _W8
        if sync:
            pltpu.sync_copy(src_hbm.at[pl.ds(rb, _W8)], sidx.at[slot])
            pltpu.sync_copy(dst_hbm.at[pl.ds(rb, _W8)], didx.at[slot])
        else:
            pltpu.async_copy(src_hbm.at[pl.ds(rb, _W8)], sidx.at[slot], semi)
            pltpu.async_copy(dst_hbm.at[pl.ds(rb, _W8)], didx.at[slot], semi)

    def _idx_drain(wave, slot):
        rb = wb + wave * _W8
        pltpu.make_async_copy(src_hbm.at[pl.ds(rb, _W8)], sidx.at[slot],
                              semi).wait()
        pltpu.make_async_copy(dst_hbm.at[pl.ds(rb, _W8)], didx.at[slot],
                              semi).wait()

    def _gather(slot, r, buf):
        pltpu.async_copy(table_hbm.at[sidx.at[slot, r]], rows.at[buf], semg)

    def _gdrain(slot, r, buf):
        pltpu.make_async_copy(table_hbm.at[sidx.at[slot, r]], rows.at[buf],
                              semg).wait()

    def _scatter(slot, r, buf):
        pltpu.sync_copy(rows.at[buf], acc.at[didx.at[slot, r]], add=True)

    @pl.when(nwv > 0)
    def _():
        _idx_load(0, 0, sync=True)
        _idx_load(1, 1)
        for b in range(4):
            _gather(0, b, b)

    def _half(w, slot):
        other = 1 - slot
        for b in range(4):
            _gather(slot, 4 + b, 4 + b)
        for b in range(4):
            _gdrain(slot, b, b)
        for b in range(4):
            _scatter(slot, b, b)

        @pl.when(w + 1 < nwv)
        def _():
            _idx_drain(w + 1, other)
            for b in range(4):
                _gather(other, b, b)

        for b in range(4):
            _gdrain(slot, 4 + b, 4 + b)
        for b in range(4):
            _scatter(slot, 4 + b, 4 + b)

        @pl.when(w + 2 < nwv)
        def _():
            _idx_load(w + 2, slot)

    @pl.loop(0, nwv, step=2)
    def _(w):
        _half(w, 0)
        _half(w + 1, 1)

    plsc.subcore_barrier()
    for k in range(_RPC // _CH):
        pltpu.sync_copy(acc.at[pl.ds(s * _RPC + k * _CH, _CH)], rows.at[0])
        pltpu.sync_copy(rows.at[0],
                        out_hbm.at[c].at[pl.ds(s * _RPC + k * _CH, _CH)])


def _tc1_body(dacc_ref, x_ref, w1_ref, g1_ref, dinv_ref):
    dacc = dacc_ref[...]
    indeg = dacc[0, :_N, 0:1] + dacc[1, :_N, 0:1]
    dinv = lax.rsqrt(indeg + 1.0)
    dinv_ref[...] = dinv
    h = jnp.dot(x_ref[...], w1_ref[...], preferred_element_type=jnp.float32)
    g1_ref[...] = dinv * h


def _tc2_body(acc_ref, g1_ref, dinv_ref, b1_ref, g2_ref):
    a = acc_ref[...]
    dinv = dinv_ref[...]
    pre = dinv * (a[0, :_N, :] + a[1, :_N, :] + g1_ref[...]) + b1_ref[...]
    h1 = jnp.maximum(pre, 0.0)
    g2_ref[...] = dinv * h1


def _tc3_body(acc_ref, g2_ref, dinv_ref, w2_ref, b2_ref, w3_ref, g3_ref):
    a = acc_ref[...]
    dinv = dinv_ref[...]
    p2 = dinv * (a[0, :_N, :] + a[1, :_N, :] + g2_ref[...])
    h2 = jnp.maximum(
        jnp.dot(p2, w2_ref[...], preferred_element_type=jnp.float32)
        + b2_ref[...], 0.0)
    g3_ref[...] = dinv * jnp.dot(h2, w3_ref[...],
                                 preferred_element_type=jnp.float32)


def _tc4_body(acc_ref, g3_ref, dinv_ref, b3_ref, batch_ref, wfc_ref, bfc_ref,
              out_ref):
    a = acc_ref[...]
    dinv = dinv_ref[...]
    pre = dinv * (a[0, :_N, :] + a[1, :_N, :] + g3_ref[...]) + b3_ref[...]
    h3 = jnp.maximum(pre, 0.0)
    seg = lax.broadcasted_iota(jnp.int32, (_G, _N), 0)
    onehot = (seg == batch_ref[...]).astype(jnp.float32)
    sums = jnp.dot(onehot, h3, preferred_element_type=jnp.float32)
    counts = jnp.sum(onehot, axis=1, keepdims=True)
    pooled = sums / jnp.maximum(counts, 1.0)
    logits = jnp.dot(pooled, wfc_ref[...],
                     preferred_element_type=jnp.float32) + bfc_ref[...]
    m = jnp.max(logits, axis=1, keepdims=True)
    shifted = logits - m
    lse = jnp.log(jnp.sum(jnp.exp(shifted), axis=1, keepdims=True))
    out_ref[...] = shifted - lse


def kernel(x, edge_index, batch, W1, b1, W2, b2, W3, b3, Wfc, bfc):
    src, dst = edge_index[0], edge_index[1]
    pad = _EPAD - _E
    pad_src = jnp.arange(pad, dtype=src.dtype) % _N
    pad_dst = _N + jnp.arange(pad, dtype=dst.dtype) % (_NPAD - _N)
    src_p = jnp.concatenate([src, pad_src])
    dst_p = jnp.concatenate([dst, pad_dst])
    src2 = src_p.reshape(_EPAD // _CH, _CH)
    dst2 = dst_p.reshape(_EPAD // _CH, _CH)
    batch2 = batch.reshape(1, _N)
    b1r, b2r, b3r = b1.reshape(1, -1), b2.reshape(1, -1), b3.reshape(1, -1)
    bfcr = bfc.reshape(1, -1)

    dacc = _sc_degree(dst2)

    g1, dinv = pl.pallas_call(
        _tc1_body,
        out_shape=(jax.ShapeDtypeStruct((_N, 64), jnp.float32),
                   jax.ShapeDtypeStruct((_N, 1), jnp.float32)),
    )(dacc, x, W1)

    a1 = _sc_propagate(g1, src2, dst2)
    g2 = pl.pallas_call(
        _tc2_body,
        out_shape=jax.ShapeDtypeStruct((_N, 64), jnp.float32),
    )(a1, g1, dinv, b1r)

    a2 = _sc_propagate(g2, src2, dst2)
    g3 = pl.pallas_call(
        _tc3_body,
        out_shape=jax.ShapeDtypeStruct((_N, 64), jnp.float32),
    )(a2, g2, dinv, W2, b2r, W3)

    a3 = _sc_propagate(g3, src2, dst2)
    out = pl.pallas_call(
        _tc4_body,
        out_shape=jax.ShapeDtypeStruct((_G, 10), jnp.float32),
    )(a3, g3, dinv, b3r, batch2, Wfc, bfcr)
    return out


# speedup vs baseline: 1.2748x; 1.2245x over previous
import functools

import jax
import jax.numpy as jnp
from jax import lax
from jax.experimental import pallas as pl
from jax.experimental.pallas import tpu as pltpu
from jax.experimental.pallas import tpu_sc as plsc

_N = 10000
_E = 320000
_G = 32
_F = 64
_NC = 2
_NS = 16
_NW = _NC * _NS
_CH = 128
_NCHUNK = 80
_CPW = _CH * _NCHUNK
_EPAD = _CPW * _NW
_TOTCH = _EPAD // _CH
_NCH0 = 80
_NCH1 = 80
_W8 = 8
_NPAD = 10240
_RPC = _NPAD // _NS
_DW = 16

_mesh = plsc.VectorSubcoreMesh(core_axis_name="c", subcore_axis_name="s")
_sc_params = pltpu.CompilerParams(use_tc_tiling_on_sc=False)


@functools.partial(
    pl.kernel,
    out_type=jax.ShapeDtypeStruct((_NC, _NPAD, _DW), jnp.float32),
    mesh=_mesh,
    compiler_params=_sc_params,
    scratch_types=[
        pltpu.VMEM((_NCHUNK, _CH), jnp.int32),
        pltpu.VMEM((_CH, _DW), jnp.float32),
        pltpu.VMEM((_RPC, _DW), jnp.float32),
        pltpu.VMEM_SHARED((_NPAD, _DW), jnp.float32),
        pltpu.SemaphoreType.DMA,
    ],
)
def _sc_degree(dst_hbm, out_hbm, didx, ones, zb, acc, sem):
    c = lax.axis_index("c")
    s = lax.axis_index("s")
    wid = c * _NS + s
    ones_v = jnp.full((16,), 1.0, jnp.float32)
    zeros_v = jnp.zeros((16,), jnp.float32)

    pltpu.sync_copy(dst_hbm.at[pl.ds(wid * _NCHUNK, _NCHUNK)], didx)

    @pl.loop(0, _CH)
    def _(i):
        ones[i, :] = ones_v

    @pl.loop(0, _RPC)
    def _(i):
        zb[i, :] = zeros_v

    pltpu.sync_copy(zb, acc.at[pl.ds(s * _RPC, _RPC)])
    plsc.subcore_barrier()

    @pl.loop(0, _NCHUNK, step=8)
    def _(j):
        for b in range(8):
            pltpu.async_copy(ones, acc.at[didx.at[j + b]], sem, add=True)
        for b in range(8):
            pltpu.make_async_copy(ones, acc.at[didx.at[j + b]], sem).wait()

    plsc.subcore_barrier()
    pltpu.sync_copy(acc.at[pl.ds(s * _RPC, _RPC)], zb)
    pltpu.sync_copy(zb, out_hbm.at[c].at[pl.ds(s * _RPC, _RPC)])


@functools.partial(
    pl.kernel,
    out_type=jax.ShapeDtypeStruct((_NC, _NPAD, _F), jnp.float32),
    mesh=_mesh,
    compiler_params=_sc_params,
    scratch_types=[
        pltpu.VMEM((2, _W8, _CH), jnp.int32),
        pltpu.VMEM((2, _W8, _CH), jnp.int32),
        pltpu.VMEM((8, _CH, _F), jnp.float32),
        pltpu.VMEM_SHARED((_NPAD, _F), jnp.float32),
        pltpu.SemaphoreType.DMA,
        pltpu.SemaphoreType.DMA,
    ],
)
def _sc_propagate(table_hbm, src_hbm, dst_hbm, out_hbm,
                  sidx, didx, rows, acc, semg, semi):
    c = lax.axis_index("c")
    s = lax.axis_index("s")
    zeros_v = jnp.zeros((16,), jnp.float32)

    wb = jnp.where(c == 0, s * _NCH0, _NS * _NCH0 + s * _NCH1)
    nwv = jnp.where(c == 0, _NCH0 // _W8, _NCH1 // _W8)

    @pl.loop(0, _CH)
    def _(i):
        for k in range(_F // 16):
            rows[0, i, pl.ds(k * 16, 16)] = zeros_v

    for k in range(_RPC // _CH):
        pltpu.sync_copy(rows.at[0], acc.at[pl.ds(s * _RPC + k * _CH, _CH)])
    plsc.subcore_barrier()

    def _idx_load(wave, slot, sync=False):
        rb = wb + wave * _W8
        if sync:
            pltpu.sync_copy(src_hbm.at[pl.ds(rb, _W8)], sidx.at[slot])
            pltpu.sync_copy(dst_hbm.at[pl.ds(rb, _W8)], didx.at[slot])
        else:
            pltpu.async_copy(src_hbm.at[pl.ds(rb, _W8)], sidx.at[slot], semi)
            pltpu.async_copy(dst_hbm.at[pl.ds(rb, _W8)], didx.at[slot], semi)

    def _idx_drain(wave, slot):
        rb = wb + wave * _W8
        pltpu.make_async_copy(src_hbm.at[pl.ds(rb, _W8)], sidx.at[slot],
                              semi).wait()
        pltpu.make_async_copy(dst_hbm.at[pl.ds(rb, _W8)], didx.at[slot],
                              semi).wait()

    def _gather(slot, r, buf):
        pltpu.async_copy(table_hbm.at[sidx.at[slot, r]], rows.at[buf], semg)

    def _gdrain(slot, r, buf):
        pltpu.make_async_copy(table_hbm.at[sidx.at[slot, r]], rows.at[buf],
                              semg).wait()

    def _scatter(slot, r, buf):
        pltpu.sync_copy(rows.at[buf], acc.at[didx.at[slot, r]], add=True)

    @pl.when(nwv > 0)
    def _():
        _idx_load(0, 0, sync=True)
        _idx_load(1, 1)
        for b in range(4):
            _gather(0, b, b)

    def _half(w, slot):
        other = 1 - slot
        for b in range(4):
            _gather(slot, 4 + b, 4 + b)
        for b in range(4):
            _gdrain(slot, b, b)
        for b in range(4):
            _scatter(slot, b, b)

        @pl.when(w + 1 < nwv)
        def _():
            _idx_drain(w + 1, other)
            for b in range(4):
                _gather(other, b, b)

        for b in range(4):
            _gdrain(slot, 4 + b, 4 + b)
        for b in range(4):
            _scatter(slot, 4 + b, 4 + b)

        @pl.when(w + 2 < nwv)
        def _():
            _idx_load(w + 2, slot)

    @pl.loop(0, nwv, step=2)
    def _(w):
        _half(w, 0)
        _half(w + 1, 1)

    plsc.subcore_barrier()
    for k in range(_RPC // _CH):
        pltpu.sync_copy(acc.at[pl.ds(s * _RPC + k * _CH, _CH)], rows.at[0])
        pltpu.sync_copy(rows.at[0],
                        out_hbm.at[c].at[pl.ds(s * _RPC + k * _CH, _CH)])


_NH = _N // 2


def _acc_pairs(acc_1d):
    v = acc_1d.reshape(_NC * _NPAD * _F // 128, 128)
    half = _NPAD * _F // 128
    return (v[:half] + v[half:])[:_NH]


def _pair(x):
    return jnp.concatenate([x[:_NH, :], x[_NH:, :]], axis=1)


def _tc1_body(dacc_ref, x_ref, w1_ref, g1p_ref, dinvp_ref):
    dacc = dacc_ref[...]
    indeg = dacc[0, :_N, 0:1] + dacc[1, :_N, 0:1]
    dinv = lax.rsqrt(indeg + 1.0)
    dinvp = _pair(jnp.broadcast_to(dinv, (_N, _F)))
    dinvp_ref[...] = dinvp
    h = jnp.dot(x_ref[...], w1_ref[...], preferred_element_type=jnp.float32)
    g1p_ref[...] = dinvp * _pair(h)


def _tc2_body(acc_ref, g1p_ref, dinvp_ref, b1p_ref, g2p_ref):
    ap = _acc_pairs(acc_ref[...])
    dinvp = dinvp_ref[...]
    pre = dinvp * (ap + g1p_ref[...]) + b1p_ref[...]
    g2p_ref[...] = dinvp * jnp.maximum(pre, 0.0)


def _tc3_body(acc_ref, g2p_ref, dinvp_ref, w2d_ref, b2p_ref, w3d_ref,
              g3p_ref):
    ap = _acc_pairs(acc_ref[...])
    dinvp = dinvp_ref[...]
    p2p = dinvp * (ap + g2p_ref[...])
    h2p = jnp.maximum(
        jnp.dot(p2p, w2d_ref[...], preferred_element_type=jnp.float32)
        + b2p_ref[...], 0.0)
    g3p = jnp.dot(h2p, w3d_ref[...], preferred_element_type=jnp.float32)
    g3p_ref[...] = dinvp * g3p


def _tc4_body(acc_ref, g3p_ref, dinvp_ref, b3p_ref, batche_ref, batcho_ref,
              wfc_ref, bfc_ref, out_ref):
    ap = _acc_pairs(acc_ref[...])
    dinvp = dinvp_ref[...]
    pre = dinvp * (ap + g3p_ref[...]) + b3p_ref[...]
    h3p = jnp.maximum(pre, 0.0)
    seg = lax.broadcasted_iota(jnp.int32, (_G, _NH), 0)
    onehot_e = (seg == batche_ref[...]).astype(jnp.float32)
    onehot_o = (seg == batcho_ref[...]).astype(jnp.float32)
    sums = (jnp.dot(onehot_e, h3p[:, :_F],
                    preferred_element_type=jnp.float32)
            + jnp.dot(onehot_o, h3p[:, _F:],
                      preferred_element_type=jnp.float32))
    counts = (jnp.sum(onehot_e, axis=1, keepdims=True)
              + jnp.sum(onehot_o, axis=1, keepdims=True))
    pooled = sums / jnp.maximum(counts, 1.0)
    logits = jnp.dot(pooled, wfc_ref[...],
                     preferred_element_type=jnp.float32) + bfc_ref[...]
    m = jnp.max(logits, axis=1, keepdims=True)
    shifted = logits - m
    lse = jnp.log(jnp.sum(jnp.exp(shifted), axis=1, keepdims=True))
    out_ref[...] = shifted - lse


def kernel(x, edge_index, batch, W1, b1, W2, b2, W3, b3, Wfc, bfc):
    src, dst = edge_index[0], edge_index[1]
    pad = _EPAD - _E
    pad_src = jnp.arange(pad, dtype=src.dtype) % _N
    pad_dst = _N + jnp.arange(pad, dtype=dst.dtype) % (_NPAD - _N)
    def _phys(i):
        return jnp.where(i < _NH, i * 2, (i - _NH) * 2 + 1)
    src_p = jnp.concatenate([_phys(src), pad_src])
    dst_p = jnp.concatenate([_phys(dst), pad_dst])
    dst_deg = jnp.concatenate([dst, pad_dst])
    src2 = src_p.reshape(_EPAD // _CH, _CH)
    dst2 = dst_p.reshape(_EPAD // _CH, _CH)
    deg2 = dst_deg.reshape(_EPAD // _CH, _CH)
    batche = batch[:_NH].reshape(1, _NH)
    batcho = batch[_NH:].reshape(1, _NH)
    b1p = jnp.concatenate([b1, b1]).reshape(1, 128)
    b3p = jnp.concatenate([b3, b3]).reshape(1, 128)
    b2p = jnp.concatenate([b2, b2]).reshape(1, 256)
    z64 = jnp.zeros((64, 128), jnp.float32)
    w2d = jnp.concatenate([jnp.concatenate([W2, z64[:, :128]], axis=1),
                           jnp.concatenate([z64[:, :128], W2], axis=1)],
                          axis=0)
    z128 = jnp.zeros((128, 64), jnp.float32)
    w3d = jnp.concatenate([jnp.concatenate([W3, z128], axis=1),
                           jnp.concatenate([z128, W3], axis=1)],
                          axis=0)
    bfcr = bfc.reshape(1, -1)
    pair_shape = jax.ShapeDtypeStruct((_NH, 128), jnp.float32)

    dacc = _sc_degree(deg2)

    g1p, dinvp = pl.pallas_call(
        _tc1_body,
        out_shape=(pair_shape, pair_shape),
    )(dacc, x, W1)

    a1 = _sc_propagate(g1p.reshape(_N, _F), src2, dst2).reshape(-1)
    g2p = pl.pallas_call(
        _tc2_body,
        out_shape=pair_shape,
    )(a1, g1p, dinvp, b1p)

    a2 = _sc_propagate(g2p.reshape(_N, _F), src2, dst2).reshape(-1)
    g3p = pl.pallas_call(
        _tc3_body,
        out_shape=pair_shape,
    )(a2, g2p, dinvp, w2d, b2p, w3d)

    a3 = _sc_propagate(g3p.reshape(_N, _F), src2, dst2).reshape(-1)
    out = pl.pallas_call(
        _tc4_body,
        out_shape=jax.ShapeDtypeStruct((_G, 10), jnp.float32),
    )(a3, g3p, dinvp, b3p, batche, batcho, Wfc, bfcr)
    return out
